# Initial kernel scaffold; baseline (speedup 1.0000x reference)
import jax
import jax.numpy as jnp
from jax.experimental import pallas as pl


def kernel(x, edge_index, edge_attr, edge_weight, W1, b1, W2, b2, eps, gamma, beta):
    raise NotImplementedError("write your pallas kernel here")


# speedup vs baseline: 3.5924x; 3.5924x over previous
import functools

import jax
import jax.numpy as jnp
from jax import lax
from jax.experimental import pallas as pl
from jax.experimental.pallas import tpu as pltpu
from jax.experimental.pallas import tpu_sc as plsc

N = 10000
E = 320000
D = 128
L = 3

NUM_CORES = 2
NUM_SUBCORES = 16
TILES = NUM_CORES * NUM_SUBCORES
CH = 128
E_PAD = ((E + TILES * CH - 1) // (TILES * CH)) * (TILES * CH)
ET = E_PAD // TILES
NCHUNK = ET // CH
ROWS_PT = (N // NUM_SUBCORES) // 8 * 8
ROWS_TAIL = N - NUM_SUBCORES * ROWS_PT


def _sc_agg_body(h_hbm, src_hbm, dst_hbm, ew_hbm, zeros_hbm, out_hbm,
                 agg_sh, src_v, dst_v, ew_v, rows_v, sem):
    c = lax.axis_index("c")
    s = lax.axis_index("s")
    tile = c * NUM_SUBCORES + s

    pltpu.sync_copy(zeros_hbm.at[pl.ds(s * ROWS_PT, ROWS_PT)],
                    agg_sh.at[pl.ds(s * ROWS_PT, ROWS_PT)])

    @pl.when(s == NUM_SUBCORES - 1)
    def _zero_tail():
        pltpu.sync_copy(zeros_hbm.at[pl.ds(NUM_SUBCORES * ROWS_PT, ROWS_TAIL)],
                        agg_sh.at[pl.ds(NUM_SUBCORES * ROWS_PT, ROWS_TAIL)])

    plsc.subcore_barrier()

    base = tile * ET

    def chunk_body(i, carry):
        off = base + i * CH
        pltpu.sync_copy(src_hbm.at[pl.ds(off, CH)], src_v)
        pltpu.sync_copy(dst_hbm.at[pl.ds(off, CH)], dst_v)
        pltpu.sync_copy(ew_hbm.at[pl.ds(off, CH)], ew_v)
        pltpu.async_copy(h_hbm.at[src_v], rows_v, sem).wait()

        def grp_body(g, carry2):
            ewg = ew_v[pl.ds(g * 16, 16)]
            for j in range(16):
                w = ewg.at[jnp.full((16,), j, dtype=jnp.int32)].get(
                    mode="promise_in_bounds", unique_indices=False)
                e = g * 16 + j
                for k in range(8):
                    rows_v[e, pl.ds(k * 16, 16)] = rows_v[e, pl.ds(k * 16, 16)] * w
            return carry2

        lax.fori_loop(0, CH // 16, grp_body, 0, unroll=False)

        pltpu.sync_copy(rows_v, agg_sh.at[dst_v], add=True)
        return carry

    lax.fori_loop(0, NCHUNK, chunk_body, 0, unroll=False)
    plsc.subcore_barrier()

    pltpu.sync_copy(agg_sh.at[pl.ds(s * ROWS_PT, ROWS_PT)],
                    out_hbm.at[c, pl.ds(s * ROWS_PT, ROWS_PT)])

    @pl.when(s == NUM_SUBCORES - 1)
    def _write_tail():
        pltpu.sync_copy(agg_sh.at[pl.ds(NUM_SUBCORES * ROWS_PT, ROWS_TAIL)],
                        out_hbm.at[c, pl.ds(NUM_SUBCORES * ROWS_PT, ROWS_TAIL)])


@jax.jit
def _sc_aggregate(h, src, dst, ew, zeros):
    mesh = plsc.VectorSubcoreMesh(core_axis_name="c", subcore_axis_name="s")
    return pl.kernel(
        _sc_agg_body,
        out_type=jax.ShapeDtypeStruct((NUM_CORES, N, D), jnp.float32),
        mesh=mesh,
        scratch_types=[
            pltpu.VMEM_SHARED((N, D), jnp.float32),
            pltpu.VMEM((CH,), jnp.int32),
            pltpu.VMEM((CH,), jnp.int32),
            pltpu.VMEM((CH,), jnp.float32),
            pltpu.VMEM((CH, D), jnp.float32),
            pltpu.SemaphoreType.DMA,
        ],
    )(h, src, dst, ew, zeros)


def _tc_dense_body(eps_ref, h_ref, agg_ref, w1_ref, b1_ref, w2_ref, b2_ref,
                   g_ref, be_ref, out_ref):
    h = h_ref[...]
    z = h * eps_ref[0] + agg_ref[0] + agg_ref[1]
    t = jnp.dot(z, w1_ref[...], preferred_element_type=jnp.float32) + b1_ref[...]
    t = jnp.maximum(t, 0.0)
    z = jnp.dot(t, w2_ref[...], preferred_element_type=jnp.float32) + b2_ref[...]
    mean = jnp.mean(z, axis=0, keepdims=True)
    var = jnp.mean(z * z, axis=0, keepdims=True) - mean * mean
    zn = (z - mean) * lax.rsqrt(var + 1e-5) * g_ref[...] + be_ref[...]
    out_ref[...] = h + jnp.maximum(zn, 0.0)


@jax.jit
def _tc_dense(eps1, h, agg, w1, b1, w2, b2, gamma, beta):
    return pl.pallas_call(
        _tc_dense_body,
        out_shape=jax.ShapeDtypeStruct((N, D), jnp.float32),
        in_specs=[
            pl.BlockSpec(memory_space=pltpu.SMEM),
            pl.BlockSpec(memory_space=pltpu.VMEM),
            pl.BlockSpec(memory_space=pltpu.VMEM),
            pl.BlockSpec(memory_space=pltpu.VMEM),
            pl.BlockSpec(memory_space=pltpu.VMEM),
            pl.BlockSpec(memory_space=pltpu.VMEM),
            pl.BlockSpec(memory_space=pltpu.VMEM),
            pl.BlockSpec(memory_space=pltpu.VMEM),
            pl.BlockSpec(memory_space=pltpu.VMEM),
        ],
        out_specs=pl.BlockSpec(memory_space=pltpu.VMEM),
    )(eps1, h, agg, w1, b1, w2, b2, gamma, beta)


def kernel(x, edge_index, edge_attr, edge_weight, W1, b1, W2, b2, eps, gamma, beta):
    del edge_attr
    src = edge_index[0]
    dst = edge_index[1]
    pad = E_PAD - E
    src_p = jnp.concatenate([src, jnp.zeros((pad,), jnp.int32)])
    dst_p = jnp.concatenate([dst, jnp.zeros((pad,), jnp.int32)])
    ew_p = jnp.concatenate([edge_weight, jnp.zeros((pad,), jnp.float32)])
    zeros = jnp.zeros((N, D), jnp.float32)

    h = x
    for i in range(L):
        agg = _sc_aggregate(h, src_p, dst_p, ew_p, zeros)
        eps1 = (1.0 + eps[i]).reshape(1)
        h = _tc_dense(eps1, h, agg,
                      W1[i], b1[i].reshape(1, D), W2[i], b2[i].reshape(1, D),
                      gamma[i].reshape(1, D), beta[i].reshape(1, D))
    return h
